# baseline (speedup 1.0000x reference)
import functools

import jax
import jax.numpy as jnp
from jax import lax
from jax.experimental import pallas as pl
from jax.experimental.pallas import tpu as pltpu
from jax.experimental.pallas import tpu_sc as plsc

TM = 96
NWORKERS = 32


def _router_body(x_ref, rw_ref, rb_ref,
                 scores_ref, w1_ref, w2_ref, d0_ref, d1_ref, counts_ref):
    x = x_ref[...]
    rw = rw_ref[...]
    logits = jax.lax.dot_general(
        x, rw, (((1,), (1,)), ((), ())), preferred_element_type=jnp.float32
    ) + rb_ref[...]
    n, e = logits.shape
    eids = jax.lax.broadcasted_iota(jnp.int32, (n, e), 1)
    l1 = jnp.max(logits, axis=1, keepdims=True)
    a1 = jnp.min(jnp.where(logits == l1, eids, e), axis=1, keepdims=True)
    oh1 = eids == a1
    masked = jnp.where(oh1, -jnp.inf, logits)
    l2 = jnp.max(masked, axis=1, keepdims=True)
    a2 = jnp.min(jnp.where(masked == l2, eids, e), axis=1, keepdims=True)
    oh2 = eids == a2
    e2 = jnp.exp(l2 - l1)
    denom = 1.0 + e2
    w1 = 1.0 / denom
    w2 = e2 / denom
    scores_ref[...] = jnp.where(oh1, w1, 0.0) + jnp.where(oh2, w2, 0.0)
    w1_ref[...] = w1
    w2_ref[...] = w2

    hist = oh1.astype(jnp.float32) + oh2.astype(jnp.float32)
    nch = n // 256
    hist_r = hist.reshape(nch, 256, e)
    tri = (jax.lax.broadcasted_iota(jnp.int32, (256, 256), 0)
           >= jax.lax.broadcasted_iota(jnp.int32, (256, 256), 1))
    trib = jnp.where(tri, 1.0, 0.0).astype(jnp.bfloat16)
    trib = jnp.broadcast_to(trib[None], (nch, 256, 256))
    csum = jax.lax.dot_general(
        trib, hist_r.astype(jnp.bfloat16),
        (((2,), (1,)), ((0,), (0,))), preferred_element_type=jnp.float32)
    chunk_tot = jax.lax.slice(csum, (0, 255, 0), (nch, 256, e)).reshape(nch, e)
    carry_rows = []
    acc = jnp.zeros((1, e), jnp.float32)
    for c in range(nch):
        carry_rows.append(acc)
        acc = acc + chunk_tot[c:c + 1, :]
    carry = jnp.concatenate(carry_rows, axis=0)
    counts = acc
    rank = (csum - hist_r + carry[:, None, :]).reshape(n, e)

    q = jnp.floor((counts + (TM - 1)) * (1.0 / TM))
    q = jnp.where(q * TM < counts, q + 1.0, q)
    q = jnp.where((q - 1.0) * TM >= counts, q - 1.0, q)
    pcount = q * TM
    incl = pcount
    k = 1
    while k < e:
        shifted = jnp.concatenate(
            [jnp.zeros((1, k), jnp.float32), incl[:, :-k]], axis=1)
        incl = incl + shifted
        k *= 2
    poffset = incl - pcount
    base = poffset + rank
    d0_ref[...] = jnp.sum(jnp.where(oh1, base, 0.0), axis=1,
                          keepdims=True).astype(jnp.int32)
    d1_ref[...] = jnp.sum(jnp.where(oh2, base, 0.0), axis=1,
                          keepdims=True).astype(jnp.int32)
    counts_ref[...] = counts.astype(jnp.int32)


def _grouped_mlp_body(te_ref, xs_ref, gw_ref, gb_ref, uw_ref, ub_ref,
                      dw_ref, db_ref, ys_ref, *, n_experts):
    i = pl.program_id(0)

    @pl.when(te_ref[i] < n_experts)
    def _work():
        xb = xs_ref[...].astype(jnp.bfloat16)
        gate = jax.lax.dot_general(
            xb, gw_ref[0].astype(jnp.bfloat16), (((1,), (0,)), ((), ())),
            preferred_element_type=jnp.float32) + gb_ref[0]
        up = jax.lax.dot_general(
            xb, uw_ref[0].astype(jnp.bfloat16), (((1,), (0,)), ((), ())),
            preferred_element_type=jnp.float32) + ub_ref[0]
        gate = jnp.minimum(gate, 7.0)
        up = jnp.clip(up, -7.0, 7.0)
        glu = gate * jax.nn.sigmoid(gate * 1.702)
        act = (up + 1.0) * glu
        ys_ref[...] = jax.lax.dot_general(
            act.astype(jnp.bfloat16), dw_ref[0].astype(jnp.bfloat16),
            (((1,), (0,)), ((), ())), preferred_element_type=jnp.float32) + db_ref[0]


def kernel(hidden_states, router_w, router_b, gate_w, gate_b, up_w, up_b,
           down_w, down_b):
    Bx, T, H = hidden_states.shape
    E, _, F = gate_w.shape
    x = hidden_states.reshape(T, H)
    nt = -(-2 * T // TM) + E
    xs_rows = nt * TM

    scores, w1, w2, d0, d1, counts = pl.pallas_call(
        _router_body,
        out_shape=[
            jax.ShapeDtypeStruct((T, E), jnp.float32),
            jax.ShapeDtypeStruct((T, 1), jnp.float32),
            jax.ShapeDtypeStruct((T, 1), jnp.float32),
            jax.ShapeDtypeStruct((T, 1), jnp.int32),
            jax.ShapeDtypeStruct((T, 1), jnp.int32),
            jax.ShapeDtypeStruct((1, E), jnp.int32),
        ],
    )(x, router_w, router_b.reshape(1, E))

    d0f = d0.reshape(T)
    d1f = d1.reshape(T)

    pt = (counts[0] + TM - 1) // TM
    cum = jnp.cumsum(pt)
    te = jnp.searchsorted(cum, jnp.arange(nt), side='right').astype(jnp.int32)

    tok_w = T // NWORKERS
    mesh = plsc.VectorSubcoreMesh(core_axis_name="c", subcore_axis_name="s")

    @functools.partial(
        pl.kernel, mesh=mesh,
        out_type=jax.ShapeDtypeStruct((xs_rows, H), jnp.float32),
        scratch_types=[
            pltpu.VMEM((tok_w,), jnp.int32),
            pltpu.VMEM((tok_w,), jnp.int32),
            pltpu.VMEM((tok_w, H), jnp.float32),
            pltpu.SemaphoreType.DMA,
            pltpu.SemaphoreType.DMA,
        ],
    )
    def _dispatch(x_hbm, d0_hbm, d1_hbm, xs_hbm, i0_v, i1_v, xbuf, sem0, sem1):
        wid = lax.axis_index("s") * 2 + lax.axis_index("c")
        rows = pl.ds(wid * tok_w, tok_w)
        pltpu.sync_copy(d0_hbm.at[rows], i0_v)
        pltpu.sync_copy(d1_hbm.at[rows], i1_v)
        pltpu.sync_copy(x_hbm.at[rows, :], xbuf)
        c0 = pltpu.async_copy(xbuf, xs_hbm.at[i0_v], sem0)
        c1 = pltpu.async_copy(xbuf, xs_hbm.at[i1_v], sem1)
        c0.wait()
        c1.wait()

    xs = _dispatch(x, d0f, d1f)

    def _emap(i, te):
        return (jnp.minimum(te[i], E - 1), 0, 0)

    def _xmap(i, te):
        return (jnp.where(te[i] < E, i, 0), 0)

    def _ymap(i, te):
        return (jnp.where(te[i] < E, i, nt - 1), 0)

    grid_spec = pltpu.PrefetchScalarGridSpec(
        num_scalar_prefetch=1,
        grid=(nt,),
        in_specs=[
            pl.BlockSpec((TM, H), _xmap),
            pl.BlockSpec((1, H, F), _emap),
            pl.BlockSpec((1, 1, F), _emap),
            pl.BlockSpec((1, H, F), _emap),
            pl.BlockSpec((1, 1, F), _emap),
            pl.BlockSpec((1, F, H), _emap),
            pl.BlockSpec((1, 1, H), _emap),
        ],
        out_specs=pl.BlockSpec((TM, H), _ymap),
    )
    ys = pl.pallas_call(
        functools.partial(_grouped_mlp_body, n_experts=E),
        grid_spec=grid_spec,
        out_shape=jax.ShapeDtypeStruct((xs_rows, H), jnp.float32),
        compiler_params=pltpu.CompilerParams(
            dimension_semantics=("arbitrary",),
        ),
    )(te, xs, gate_w, gate_b.reshape(E, 1, F), up_w, up_b.reshape(E, 1, F),
      down_w, down_b.reshape(E, 1, H))

    ch = tok_w // 2
    nsl = H // 16

    @functools.partial(
        pl.kernel, mesh=mesh,
        out_type=jax.ShapeDtypeStruct((T, H), jnp.float32),
        scratch_types=[
            pltpu.VMEM((ch,), jnp.int32),
            pltpu.VMEM((ch,), jnp.int32),
            pltpu.VMEM((ch,), jnp.float32),
            pltpu.VMEM((ch,), jnp.float32),
            pltpu.VMEM((ch, H), jnp.float32),
            pltpu.VMEM((ch, H), jnp.float32),
            pltpu.SemaphoreType.DMA,
            pltpu.SemaphoreType.DMA,
        ],
    )
    def _combine(ys_hbm, d0_hbm, d1_hbm, w1_hbm, w2_hbm, out_hbm,
                 i0_v, i1_v, wa_v, wb_v, buf0, buf1, sem0, sem1):
        wid = lax.axis_index("s") * 2 + lax.axis_index("c")
        for c in range(tok_w // ch):
            rows = pl.ds(wid * tok_w + c * ch, ch)
            pltpu.sync_copy(d0_hbm.at[rows], i0_v)
            pltpu.sync_copy(d1_hbm.at[rows], i1_v)
            pltpu.sync_copy(w1_hbm.at[rows], wa_v)
            pltpu.sync_copy(w2_hbm.at[rows], wb_v)
            cp0 = pltpu.async_copy(ys_hbm.at[i0_v], buf0, sem0)
            cp1 = pltpu.async_copy(ys_hbm.at[i1_v], buf1, sem1)
            cp0.wait()
            cp1.wait()

            def _row(r, _):
                grp = pl.ds((r // 16) * 16, 16)
                lane = jnp.full((16,), r % 16, jnp.int32)
                wa = wa_v[grp].at[lane].get(mode='promise_in_bounds')
                wb = wb_v[grp].at[lane].get(mode='promise_in_bounds')
                for j in range(nsl):
                    sl = pl.ds(j * 16, 16)
                    buf0[r, sl] = wa * buf0[r, sl] + wb * buf1[r, sl]
                return 0

            lax.fori_loop(0, ch, _row, 0)
            pltpu.sync_copy(buf0, out_hbm.at[rows, :])

    out = _combine(ys, d0f, d1f, w1.reshape(T), w2.reshape(T))

    return out.reshape(Bx, T, H), scores


# speedup vs baseline: 1.0108x; 1.0108x over previous
import functools

import jax
import jax.numpy as jnp
from jax import lax
from jax.experimental import pallas as pl
from jax.experimental.pallas import tpu as pltpu
from jax.experimental.pallas import tpu_sc as plsc

TM = 96
NWORKERS = 32


def _router_body(x_ref, rw_ref, rb_ref,
                 scores_ref, w1_ref, w2_ref, d0_ref, d1_ref, counts_ref):
    x = x_ref[...]
    rw = rw_ref[...]
    logits = jax.lax.dot_general(
        x, rw, (((1,), (1,)), ((), ())), preferred_element_type=jnp.float32
    ) + rb_ref[...]
    n, e = logits.shape
    eids = jax.lax.broadcasted_iota(jnp.int32, (n, e), 1)
    l1 = jnp.max(logits, axis=1, keepdims=True)
    a1 = jnp.min(jnp.where(logits == l1, eids, e), axis=1, keepdims=True)
    oh1 = eids == a1
    masked = jnp.where(oh1, -jnp.inf, logits)
    l2 = jnp.max(masked, axis=1, keepdims=True)
    a2 = jnp.min(jnp.where(masked == l2, eids, e), axis=1, keepdims=True)
    oh2 = eids == a2
    e2 = jnp.exp(l2 - l1)
    denom = 1.0 + e2
    w1 = 1.0 / denom
    w2 = e2 / denom
    scores_ref[...] = jnp.where(oh1, w1, 0.0) + jnp.where(oh2, w2, 0.0)
    w1_ref[...] = w1
    w2_ref[...] = w2

    hist = oh1.astype(jnp.float32) + oh2.astype(jnp.float32)
    nch = n // 256
    hist_r = hist.reshape(nch, 256, e)
    tri = (jax.lax.broadcasted_iota(jnp.int32, (256, 256), 0)
           >= jax.lax.broadcasted_iota(jnp.int32, (256, 256), 1))
    trib = jnp.where(tri, 1.0, 0.0).astype(jnp.bfloat16)
    trib = jnp.broadcast_to(trib[None], (nch, 256, 256))
    csum = jax.lax.dot_general(
        trib, hist_r.astype(jnp.bfloat16),
        (((2,), (1,)), ((0,), (0,))), preferred_element_type=jnp.float32)
    chunk_tot = jax.lax.slice(csum, (0, 255, 0), (nch, 256, e)).reshape(nch, e)
    carry_rows = []
    acc = jnp.zeros((1, e), jnp.float32)
    for c in range(nch):
        carry_rows.append(acc)
        acc = acc + chunk_tot[c:c + 1, :]
    carry = jnp.concatenate(carry_rows, axis=0)
    counts = acc
    rank = (csum - hist_r + carry[:, None, :]).reshape(n, e)

    q = jnp.floor((counts + (TM - 1)) * (1.0 / TM))
    q = jnp.where(q * TM < counts, q + 1.0, q)
    q = jnp.where((q - 1.0) * TM >= counts, q - 1.0, q)
    pcount = q * TM
    incl = pcount
    k = 1
    while k < e:
        shifted = jnp.concatenate(
            [jnp.zeros((1, k), jnp.float32), incl[:, :-k]], axis=1)
        incl = incl + shifted
        k *= 2
    poffset = incl - pcount
    base = poffset + rank
    d0_ref[...] = jnp.sum(jnp.where(oh1, base, 0.0), axis=1,
                          keepdims=True).astype(jnp.int32)
    d1_ref[...] = jnp.sum(jnp.where(oh2, base, 0.0), axis=1,
                          keepdims=True).astype(jnp.int32)
    counts_ref[...] = counts.astype(jnp.int32)


def _grouped_mlp_body(te_ref, xs_ref, gw_ref, gb_ref, uw_ref, ub_ref,
                      dw_ref, db_ref, ys_ref, *, n_experts):
    i = pl.program_id(0)

    @pl.when(te_ref[i] < n_experts)
    def _work():
        xb = xs_ref[...].astype(jnp.bfloat16)
        gate = jax.lax.dot_general(
            xb, gw_ref[0].astype(jnp.bfloat16), (((1,), (0,)), ((), ())),
            preferred_element_type=jnp.float32) + gb_ref[0]
        up = jax.lax.dot_general(
            xb, uw_ref[0].astype(jnp.bfloat16), (((1,), (0,)), ((), ())),
            preferred_element_type=jnp.float32) + ub_ref[0]
        gate = jnp.minimum(gate, 7.0)
        up = jnp.clip(up, -7.0, 7.0)
        glu = gate * jax.nn.sigmoid(gate * 1.702)
        act = (up + 1.0) * glu
        ys_ref[...] = jax.lax.dot_general(
            act.astype(jnp.bfloat16), dw_ref[0].astype(jnp.bfloat16),
            (((1,), (0,)), ((), ())), preferred_element_type=jnp.float32) + db_ref[0]


def kernel(hidden_states, router_w, router_b, gate_w, gate_b, up_w, up_b,
           down_w, down_b):
    Bx, T, H = hidden_states.shape
    E, _, F = gate_w.shape
    x = hidden_states.reshape(T, H)
    nt = -(-2 * T // TM) + E
    xs_rows = nt * TM

    scores, w1, w2, d0, d1, counts = pl.pallas_call(
        _router_body,
        out_shape=[
            jax.ShapeDtypeStruct((T, E), jnp.float32),
            jax.ShapeDtypeStruct((T, 1), jnp.float32),
            jax.ShapeDtypeStruct((T, 1), jnp.float32),
            jax.ShapeDtypeStruct((T, 1), jnp.int32),
            jax.ShapeDtypeStruct((T, 1), jnp.int32),
            jax.ShapeDtypeStruct((1, E), jnp.int32),
        ],
    )(x, router_w, router_b.reshape(1, E))

    d0f = d0.reshape(T)
    d1f = d1.reshape(T)

    pt = (counts[0] + TM - 1) // TM
    cum = jnp.cumsum(pt)
    te = jnp.searchsorted(cum, jnp.arange(nt), side='right').astype(jnp.int32)

    tok_w = T // NWORKERS
    mesh = plsc.VectorSubcoreMesh(core_axis_name="c", subcore_axis_name="s")

    @functools.partial(
        pl.kernel, mesh=mesh,
        out_type=jax.ShapeDtypeStruct((xs_rows, H), jnp.float32),
        scratch_types=[
            pltpu.VMEM((tok_w,), jnp.int32),
            pltpu.VMEM((tok_w,), jnp.int32),
            pltpu.VMEM((tok_w, H), jnp.float32),
            pltpu.SemaphoreType.DMA,
            pltpu.SemaphoreType.DMA,
        ],
    )
    def _dispatch(x_hbm, d0_hbm, d1_hbm, xs_hbm, i0_v, i1_v, xbuf, sem0, sem1):
        wid = lax.axis_index("s") * 2 + lax.axis_index("c")
        rows = pl.ds(wid * tok_w, tok_w)
        pltpu.sync_copy(d0_hbm.at[rows], i0_v)
        pltpu.sync_copy(d1_hbm.at[rows], i1_v)
        pltpu.sync_copy(x_hbm.at[rows, :], xbuf)
        c0 = pltpu.async_copy(xbuf, xs_hbm.at[i0_v], sem0)
        c1 = pltpu.async_copy(xbuf, xs_hbm.at[i1_v], sem1)
        c0.wait()
        c1.wait()

    xs = _dispatch(x, d0f, d1f)

    def _emap(i, te):
        return (jnp.minimum(te[i], E - 1), 0, 0)

    def _xmap(i, te):
        return (jnp.where(te[i] < E, i, 0), 0)

    def _ymap(i, te):
        return (jnp.where(te[i] < E, i, nt - 1), 0)

    grid_spec = pltpu.PrefetchScalarGridSpec(
        num_scalar_prefetch=1,
        grid=(nt,),
        in_specs=[
            pl.BlockSpec((TM, H), _xmap),
            pl.BlockSpec((1, H, F), _emap),
            pl.BlockSpec((1, 1, F), _emap),
            pl.BlockSpec((1, H, F), _emap),
            pl.BlockSpec((1, 1, F), _emap),
            pl.BlockSpec((1, F, H), _emap),
            pl.BlockSpec((1, 1, H), _emap),
        ],
        out_specs=pl.BlockSpec((TM, H), _ymap),
    )
    ys = pl.pallas_call(
        functools.partial(_grouped_mlp_body, n_experts=E),
        grid_spec=grid_spec,
        out_shape=jax.ShapeDtypeStruct((xs_rows, H), jnp.float32),
        compiler_params=pltpu.CompilerParams(
            dimension_semantics=("arbitrary",),
        ),
    )(te, xs, gate_w, gate_b.reshape(E, 1, F), up_w, up_b.reshape(E, 1, F),
      down_w, down_b.reshape(E, 1, H))

    ch = tok_w // 4
    nch_c = tok_w // ch
    nsl = H // 16

    @functools.partial(
        pl.kernel, mesh=mesh,
        out_type=jax.ShapeDtypeStruct((T, H), jnp.float32),
        scratch_types=[
            [pltpu.VMEM((ch,), jnp.int32) for _ in range(2)],
            [pltpu.VMEM((ch,), jnp.int32) for _ in range(2)],
            pltpu.VMEM((ch,), jnp.float32),
            pltpu.VMEM((ch,), jnp.float32),
            [pltpu.VMEM((ch, H), jnp.float32) for _ in range(2)],
            [pltpu.VMEM((ch, H), jnp.float32) for _ in range(2)],
            [pltpu.SemaphoreType.DMA for _ in range(2)],
            [pltpu.SemaphoreType.DMA for _ in range(2)],
        ],
    )
    def _combine(ys_hbm, d0_hbm, d1_hbm, w1_hbm, w2_hbm, out_hbm,
                 i0_v, i1_v, wa_v, wb_v, buf0, buf1, sem0, sem1):
        wid = lax.axis_index("s") * 2 + lax.axis_index("c")

        def _issue(c, slot):
            rows = pl.ds(wid * tok_w + c * ch, ch)
            pltpu.sync_copy(d0_hbm.at[rows], i0_v[slot])
            pltpu.sync_copy(d1_hbm.at[rows], i1_v[slot])
            return (pltpu.async_copy(ys_hbm.at[i0_v[slot]], buf0[slot], sem0[slot]),
                    pltpu.async_copy(ys_hbm.at[i1_v[slot]], buf1[slot], sem1[slot]))

        pend = _issue(0, 0)
        for c in range(nch_c):
            slot = c % 2
            rows = pl.ds(wid * tok_w + c * ch, ch)
            pltpu.sync_copy(w1_hbm.at[rows], wa_v)
            pltpu.sync_copy(w2_hbm.at[rows], wb_v)
            nxt = _issue(c + 1, 1 - slot) if c + 1 < nch_c else None
            pend[0].wait()
            pend[1].wait()

            def _row(r, _):
                grp = pl.ds((r // 16) * 16, 16)
                lane = jnp.full((16,), r % 16, jnp.int32)
                wa = wa_v[grp].at[lane].get(mode='promise_in_bounds')
                wb = wb_v[grp].at[lane].get(mode='promise_in_bounds')
                for j in range(nsl):
                    sl = pl.ds(j * 16, 16)
                    buf0[slot][r, sl] = (wa * buf0[slot][r, sl]
                                         + wb * buf1[slot][r, sl])
                return 0

            lax.fori_loop(0, ch, _row, 0)
            pltpu.sync_copy(buf0[slot], out_hbm.at[rows, :])
            pend = nxt

    out = _combine(ys, d0f, d1f, w1.reshape(T), w2.reshape(T))

    return out.reshape(Bx, T, H), scores
